# baseline (speedup 1.0000x reference)
import functools

import jax
import jax.numpy as jnp
from jax import lax
from jax.experimental import pallas as pl
from jax.experimental.pallas import tpu as pltpu
from jax.experimental.pallas import tpu_sc as plsc

N = 10000
E = 320000
D = 128
NC = 2
NS = 16
NPAD = 10240

_MESH = dict(core_axis_name="c", subcore_axis_name="s",
             num_cores=NC, num_subcores=NS)


_EPW = E // (NC * NS)
_RPW = NPAD // NS
_DCH = 64
_DNCH = _EPW // _DCH
_DTAIL = _EPW - _DNCH * _DCH
_W = 16
_BCR = 8


def _deg_body(col_hbm, out_hbm, cbuf, cbuf_t, ones, ones_t, degv, bcast, shared):
    c = lax.axis_index("c")
    s = lax.axis_index("s")
    wid = c * NS + s
    zeros16 = jnp.zeros((_W,), jnp.float32)
    ones16 = jnp.ones((_W,), jnp.float32)

    def fill_ones(i, _):
        ones[i, pl.ds(0, _W)] = ones16
        return 0

    lax.fori_loop(0, _DCH, fill_ones, 0)

    def fill_ones_t(i, _):
        ones_t[i, pl.ds(0, _W)] = ones16
        return 0

    lax.fori_loop(0, _DTAIL, fill_ones_t, 0)

    def zero_body(i, _):
        degv[i, pl.ds(0, _W)] = zeros16
        return 0

    lax.fori_loop(0, _BCR, zero_body, 0)

    def zcopy(k, _):
        pltpu.sync_copy(degv, shared.at[pl.ds(s * _RPW + k * _BCR, _BCR)])
        return 0

    lax.fori_loop(0, _RPW // _BCR, zcopy, 0)
    plsc.subcore_barrier()

    def blk_body(b, _):
        pltpu.sync_copy(col_hbm.at[pl.ds(wid * _EPW + b * _DCH, _DCH)], cbuf)
        pltpu.sync_copy(ones, shared.at[cbuf], add=True)
        return 0

    lax.fori_loop(0, _DNCH, blk_body, 0)
    pltpu.sync_copy(col_hbm.at[pl.ds(wid * _EPW + _DNCH * _DCH, _DTAIL)], cbuf_t)
    pltpu.sync_copy(ones_t, shared.at[cbuf_t], add=True)
    plsc.subcore_barrier()

    def stage_body(k, _):
        pltpu.sync_copy(shared.at[pl.ds(s * _RPW + k * _BCR, _BCR)], degv)

        def bc_body(i, _):
            v = degv[i, pl.ds(0, _W)]
            for j in range(D // _W):
                bcast[i, pl.ds(j * _W, _W)] = v
            return 0

        lax.fori_loop(0, _BCR, bc_body, 0)
        pltpu.sync_copy(bcast,
                        out_hbm.at[pl.ds(c * NPAD + s * _RPW + k * _BCR, _BCR)])
        return 0

    lax.fori_loop(0, _RPW // _BCR, stage_body, 0)


@functools.cache
def _build_deg():
    return pl.kernel(
        _deg_body,
        mesh=plsc.VectorSubcoreMesh(**_MESH),
        compiler_params=pltpu.CompilerParams(needs_layout_passes=False, use_tc_tiling_on_sc=False),
        out_type=jax.ShapeDtypeStruct((NC * NPAD, D), jnp.float32),
        scratch_types=[
            pltpu.VMEM((_DCH,), jnp.int32),
            pltpu.VMEM((_DTAIL,), jnp.int32),
            pltpu.VMEM((_DCH, _W), jnp.float32),
            pltpu.VMEM((_DTAIL, _W), jnp.float32),
            pltpu.VMEM((_BCR, _W), jnp.float32),
            pltpu.VMEM((_BCR, D), jnp.float32),
            pltpu.VMEM_SHARED((NPAD, _W), jnp.float32),
        ],
    )


def _deg_kernel(col):
    return _build_deg()(col)


_CH = 96
_CPB = 30
_NBLK = 7
_CPT = _CPB * _NBLK
_EPT = _CPT * _CH
_EPAD = NS * _EPT
_ROWS_PW = NPAD // NS
_HW = D // 2


def _agg_body(row_hbm, col_hbm, g00, g01, g10, g11, out0_hbm, out1_hbm,
              ridxb, cidxb, rows0, rows1, accum, sem0, sem1, ssem0, ssem1):
    c = lax.axis_index("c")
    s = lax.axis_index("s")
    zeros16 = jnp.zeros((16,), jnp.float32)

    def gather(g, j, rows, sem):
        pltpu.async_copy(g.at[ridxb.at[j]], rows, sem)

    def gwait(g, rows, sem):
        pltpu.make_async_copy(g.at[pl.ds(0, _CH)], rows, sem).wait()

    def scatter(j, rows, sem):
        pltpu.async_copy(rows, accum.at[cidxb.at[j]], sem, add=True)

    def swait(g, rows, sem):
        pltpu.make_async_copy(g.at[pl.ds(0, _CH)], rows, sem).wait()

    for half in range(2):
        ga = g00 if half == 0 else g01
        gb = g10 if half == 0 else g11
        out = out0_hbm if half == 0 else out1_hbm

        def zr(i, _):
            for j in range(_HW // 16):
                rows0[i, pl.ds(j * 16, 16)] = zeros16
            return 0

        lax.fori_loop(0, _CH, zr, 0)

        def zacc(k, _):
            pltpu.sync_copy(rows0, accum.at[pl.ds(s * _ROWS_PW + k * _CH, _CH)])
            return 0

        lax.fori_loop(0, _ROWS_PW // _CH, zacc, 0)
        pltpu.sync_copy(rows0.at[pl.ds(0, _ROWS_PW - (_ROWS_PW // _CH) * _CH)],
                        accum.at[pl.ds(s * _ROWS_PW + (_ROWS_PW // _CH) * _CH,
                                       _ROWS_PW - (_ROWS_PW // _CH) * _CH)])
        plsc.subcore_barrier()

        def blk_body(b, _):
            base = s * _CPT + b * _CPB
            pltpu.sync_copy(row_hbm.at[pl.ds(base, _CPB)], ridxb)
            pltpu.sync_copy(col_hbm.at[pl.ds(base, _CPB)], cidxb)

            def run(g):
                gather(g, 0, rows0, sem0)
                gather(g, 1, rows1, sem1)

                def pair(p, _):
                    j = 2 * p
                    gwait(g, rows0, sem0)
                    scatter(j, rows0, ssem0)
                    gwait(g, rows1, sem1)
                    scatter(j + 1, rows1, ssem1)
                    swait(g, rows0, ssem0)
                    gather(g, j + 2, rows0, sem0)
                    swait(g, rows1, ssem1)
                    gather(g, j + 3, rows1, sem1)
                    return 0

                lax.fori_loop(0, _CPB // 2 - 1, pair, 0)
                gwait(g, rows0, sem0)
                scatter(_CPB - 2, rows0, ssem0)
                gwait(g, rows1, sem1)
                scatter(_CPB - 1, rows1, ssem1)
                swait(g, rows0, ssem0)
                swait(g, rows1, ssem1)

            @pl.when(c == 0)
            def _():
                run(ga)

            @pl.when(c == 1)
            def _():
                run(gb)

            return 0

        lax.fori_loop(0, _NBLK, blk_body, 0)

        plsc.subcore_barrier()
        pltpu.sync_copy(accum.at[pl.ds(s * _ROWS_PW, _ROWS_PW)],
                        out.at[pl.ds(c * NPAD + s * _ROWS_PW, _ROWS_PW)])
        plsc.subcore_barrier()


@functools.cache
def _build_agg():
    return pl.kernel(
        _agg_body,
        mesh=plsc.VectorSubcoreMesh(**_MESH),
        compiler_params=pltpu.CompilerParams(needs_layout_passes=False, use_tc_tiling_on_sc=False),
        out_type=(jax.ShapeDtypeStruct((NC * NPAD, _HW), jnp.float32),
                  jax.ShapeDtypeStruct((NC * NPAD, _HW), jnp.float32)),
        scratch_types=[
            pltpu.VMEM((_CPB, _CH), jnp.int32),
            pltpu.VMEM((_CPB, _CH), jnp.int32),
            pltpu.VMEM((_CH, _HW), jnp.float32),
            pltpu.VMEM((_CH, _HW), jnp.float32),
            pltpu.VMEM_SHARED((NPAD, _HW), jnp.float32),
            pltpu.SemaphoreType.DMA,
            pltpu.SemaphoreType.DMA,
            pltpu.SemaphoreType.DMA,
            pltpu.SemaphoreType.DMA,
        ],
    )


def _agg_kernel(row2d, col2d, g00, g01, g10, g11):
    return _build_agg()(row2d, col2d, g00, g01, g10, g11)


_BR = 2000
_GRID = N // _BR


def _rowspec(w=D):
    return pl.BlockSpec((_BR, w), lambda i: (i, 0))


def _fullspec(shape):
    return pl.BlockSpec(shape, lambda i: tuple(0 for _ in shape))


def _tc1_body(d0_ref, d1_ref, x_ref, wc_ref,
              g00_ref, g01_ref, g10_ref, g11_ref, dinv_ref):
    deg = d0_ref[...] + d1_ref[...]
    dinv = jnp.where(deg > 0, lax.rsqrt(jnp.maximum(deg, 1e-12)), 0.0)
    dinv_ref[...] = dinv
    h = jnp.dot(x_ref[...], wc_ref[...], preferred_element_type=jnp.float32,
                precision=lax.Precision.HIGHEST)
    ge = h[:, :D] * dinv
    gd = h[:, D:] * dinv
    g00_ref[...] = ge[:, :_HW]
    g01_ref[...] = ge[:, _HW:]
    g10_ref[...] = gd[:, :_HW]
    g11_ref[...] = gd[:, _HW:]


def _tc1(d0, d1, x, wc):
    return pl.pallas_call(
        _tc1_body,
        grid=(_GRID,),
        in_specs=[_rowspec(), _rowspec(), _rowspec(), _fullspec((D, 2 * D))],
        out_specs=(_rowspec(_HW), _rowspec(_HW), _rowspec(_HW), _rowspec(_HW),
                   _rowspec()),
        out_shape=(jax.ShapeDtypeStruct((N, _HW), jnp.float32),
                   jax.ShapeDtypeStruct((N, _HW), jnp.float32),
                   jax.ShapeDtypeStruct((N, _HW), jnp.float32),
                   jax.ShapeDtypeStruct((N, _HW), jnp.float32),
                   jax.ShapeDtypeStruct((N, D), jnp.float32)),
        compiler_params=pltpu.CompilerParams(
            dimension_semantics=("arbitrary",)),
    )(d0, d1, x, wc)


def _tc2_body(s0l_ref, s0h_ref, s1l_ref, s1h_ref, dinv_ref,
              b1e_ref, b1d_ref, w2e_ref, w2d_ref,
              g00_ref, g01_ref, g10_ref, g11_ref, x1e_ref, x1d_ref):
    dinv = dinv_ref[...]
    s0 = jnp.concatenate([s0l_ref[...], s0h_ref[...]], axis=1)
    s1 = jnp.concatenate([s1l_ref[...], s1h_ref[...]], axis=1)
    x1e = jnp.tanh(s0 * dinv + b1e_ref[...])
    x1d = jnp.tanh(s1 * dinv + b1d_ref[...])
    x1e_ref[...] = x1e
    x1d_ref[...] = x1d
    ge = dinv * jnp.dot(x1e, w2e_ref[...], preferred_element_type=jnp.float32,
                        precision=lax.Precision.HIGHEST)
    gd = dinv * jnp.dot(x1d, w2d_ref[...], preferred_element_type=jnp.float32,
                        precision=lax.Precision.HIGHEST)
    g00_ref[...] = ge[:, :_HW]
    g01_ref[...] = ge[:, _HW:]
    g10_ref[...] = gd[:, :_HW]
    g11_ref[...] = gd[:, _HW:]


def _tc2(s0l, s0h, s1l, s1h, dinv, b1e, b1d, w2e, w2d):
    return pl.pallas_call(
        _tc2_body,
        grid=(_GRID,),
        in_specs=[_rowspec(_HW), _rowspec(_HW), _rowspec(_HW), _rowspec(_HW),
                  _rowspec(),
                  _fullspec((1, D)), _fullspec((1, D)),
                  _fullspec((D, D)), _fullspec((D, D))],
        out_specs=(_rowspec(_HW), _rowspec(_HW), _rowspec(_HW), _rowspec(_HW),
                   _rowspec(), _rowspec()),
        out_shape=(jax.ShapeDtypeStruct((N, _HW), jnp.float32),
                   jax.ShapeDtypeStruct((N, _HW), jnp.float32),
                   jax.ShapeDtypeStruct((N, _HW), jnp.float32),
                   jax.ShapeDtypeStruct((N, _HW), jnp.float32),
                   jax.ShapeDtypeStruct((N, D), jnp.float32),
                   jax.ShapeDtypeStruct((N, D), jnp.float32)),
        compiler_params=pltpu.CompilerParams(
            dimension_semantics=("arbitrary",)),
    )(s0l, s0h, s1l, s1h, dinv, b1e, b1d, w2e, w2d)


def _layernorm(y, w, b, eps=1e-5):
    mu = jnp.mean(y, axis=-1, keepdims=True)
    var = jnp.mean((y - mu) ** 2, axis=-1, keepdims=True)
    return (y - mu) / jnp.sqrt(var + eps) * w + b


def _tc3_body(s0l_ref, s0h_ref, s1l_ref, s1h_ref, dinv_ref, x1e_ref, x1d_ref,
              b2e_ref, b2d_ref, lwe_ref, lbe_ref, lwd_ref, lbd_ref,
              lnwe_ref, lnbe_ref, lnwd_ref, lnbd_ref,
              bnw_ref, bnb_ref, fw_ref, out_ref):
    dinv = dinv_ref[...]
    s0 = jnp.concatenate([s0l_ref[...], s0h_ref[...]], axis=1)
    s1 = jnp.concatenate([s1l_ref[...], s1h_ref[...]], axis=1)
    x2e = jnp.tanh(s0 * dinv + b2e_ref[...])
    x2d = jnp.tanh(s1 * dinv + b2d_ref[...])
    xme = jnp.maximum(x1e_ref[...], x2e)
    xmd = jnp.maximum(x1d_ref[...], x2d)
    ye = jnp.dot(xme, lwe_ref[...], preferred_element_type=jnp.float32,
                 precision=lax.Precision.HIGHEST) + lbe_ref[...]
    yd = jnp.dot(xmd, lwd_ref[...], preferred_element_type=jnp.float32,
                 precision=lax.Precision.HIGHEST) + lbd_ref[...]
    lne = _layernorm(ye, lnwe_ref[...], lnbe_ref[...])
    lnd = _layernorm(yd, lnwd_ref[...], lnbd_ref[...])
    node = (lne + lnd) * 0.5
    node = node / jnp.sqrt(1.0 + 1e-5) * bnw_ref[...] + bnb_ref[...]
    out_ref[...] = jnp.sum(node * fw_ref[...], axis=1).reshape(1, 8, _BR // 8)


def _tc3(s0l, s0h, s1l, s1h, dinv, x1e, x1d, b2e, b2d, lwe, lbe, lwd, lbd,
         lnwe, lnbe, lnwd, lnbd, bnw, bnb, fw):
    return pl.pallas_call(
        _tc3_body,
        grid=(_GRID,),
        in_specs=[_rowspec(_HW), _rowspec(_HW), _rowspec(_HW), _rowspec(_HW),
                  _rowspec(), _rowspec(), _rowspec(),
                  _fullspec((1, D)), _fullspec((1, D)),
                  _fullspec((D, D)), _fullspec((1, D)),
                  _fullspec((D, D)), _fullspec((1, D)),
                  _fullspec((1, D)), _fullspec((1, D)),
                  _fullspec((1, D)), _fullspec((1, D)),
                  _fullspec((1, D)), _fullspec((1, D)),
                  _fullspec((1, D))],
        out_specs=pl.BlockSpec((1, 8, _BR // 8), lambda i: (i, 0, 0)),
        out_shape=jax.ShapeDtypeStruct((_GRID, 8, _BR // 8), jnp.float32),
        compiler_params=pltpu.CompilerParams(
            dimension_semantics=("arbitrary",)),
    )(s0l, s0h, s1l, s1h, dinv, x1e, x1d, b2e, b2d, lwe, lbe, lwd, lbd,
      lnwe, lnbe, lnwd, lnbd, bnw, bnb, fw).reshape(N)


def kernel(x, edge_index, enc_W1, enc_b1, enc_W2, enc_b2, enc_linW, enc_linb,
           dec_W1, dec_b1, dec_W2, dec_b2, dec_linW, dec_linb,
           enc_ln_w, enc_ln_b, dec_ln_w, dec_ln_b, bn_w, bn_b, final_W):
    row = edge_index[0]
    col = edge_index[1]
    npad_e = _EPAD - E
    row_p = jnp.concatenate(
        [row, jnp.zeros((npad_e,), jnp.int32)]).reshape(NS * _CPT, _CH)
    col_p = jnp.concatenate(
        [col, jnp.full((npad_e,), N, jnp.int32)]).reshape(NS * _CPT, _CH)

    degp = _deg_kernel(col)
    d0 = degp[:N]
    d1 = degp[NPAD:NPAD + N]

    wc = jnp.concatenate([enc_W1, dec_W1], axis=1)
    g00, g01, g10, g11, dinv = _tc1(d0, d1, x, wc)

    r1 = lambda v: v.reshape(1, D)
    s1l, s1h = _agg_kernel(row_p, col_p, g00, g01, g10, g11)
    h00, h01, h10, h11, x1e, x1d = _tc2(
        s1l[:N], s1h[:N], s1l[NPAD:NPAD + N], s1h[NPAD:NPAD + N], dinv,
        r1(enc_b1), r1(dec_b1), enc_W2, dec_W2)

    s2l, s2h = _agg_kernel(row_p, col_p, h00, h01, h10, h11)
    return _tc3(s2l[:N], s2h[:N], s2l[NPAD:NPAD + N], s2h[NPAD:NPAD + N],
                dinv, x1e, x1d,
                r1(enc_b2), r1(dec_b2), enc_linW, r1(enc_linb),
                dec_linW, r1(dec_linb),
                r1(enc_ln_w), r1(enc_ln_b), r1(dec_ln_w), r1(dec_ln_b),
                r1(bn_w), r1(bn_b), final_W.reshape(1, D))


# speedup vs baseline: 1.1316x; 1.1316x over previous
import functools

import jax
import jax.numpy as jnp
from jax import lax
from jax.experimental import pallas as pl
from jax.experimental.pallas import tpu as pltpu
from jax.experimental.pallas import tpu_sc as plsc

N = 10000
E = 320000
D = 128
NC = 2
NS = 16
NPAD = 10240

_MESH = dict(core_axis_name="c", subcore_axis_name="s",
             num_cores=NC, num_subcores=NS)


_EPW = E // (NC * NS)
_RPW = NPAD // NS
_DCH = 64
_DNCH = _EPW // _DCH
_DTAIL = _EPW - _DNCH * _DCH
_W = 16
_ZR = 64


def _deg_body(col_hbm, out_hbm, cbuf, cbuf_t, ones, ones_t, zbuf, shared):
    c = lax.axis_index("c")
    s = lax.axis_index("s")
    wid = c * NS + s
    zeros16 = jnp.zeros((_W,), jnp.float32)
    ones16 = jnp.ones((_W,), jnp.float32)

    def fill_ones(i, _):
        ones[i, pl.ds(0, _W)] = ones16
        return 0

    lax.fori_loop(0, _DCH, fill_ones, 0)

    def fill_ones_t(i, _):
        ones_t[i, pl.ds(0, _W)] = ones16
        return 0

    lax.fori_loop(0, _DTAIL, fill_ones_t, 0)

    def zero_body(i, _):
        zbuf[i, pl.ds(0, _W)] = zeros16
        return 0

    lax.fori_loop(0, _ZR, zero_body, 0)

    def zcopy(k, _):
        pltpu.sync_copy(zbuf, shared.at[pl.ds(s * _RPW + k * _ZR, _ZR)])
        return 0

    lax.fori_loop(0, _RPW // _ZR, zcopy, 0)
    plsc.subcore_barrier()

    def blk_body(b, _):
        pltpu.sync_copy(col_hbm.at[pl.ds(wid * _EPW + b * _DCH, _DCH)], cbuf)
        pltpu.sync_copy(ones, shared.at[cbuf], add=True)
        return 0

    lax.fori_loop(0, _DNCH, blk_body, 0)
    pltpu.sync_copy(col_hbm.at[pl.ds(wid * _EPW + _DNCH * _DCH, _DTAIL)], cbuf_t)
    pltpu.sync_copy(ones_t, shared.at[cbuf_t], add=True)
    plsc.subcore_barrier()

    pltpu.sync_copy(shared.at[pl.ds(s * _RPW, _RPW)],
                    out_hbm.at[pl.ds(c * NPAD + s * _RPW, _RPW)])


@functools.cache
def _build_deg():
    return pl.kernel(
        _deg_body,
        mesh=plsc.VectorSubcoreMesh(**_MESH),
        compiler_params=pltpu.CompilerParams(needs_layout_passes=False, use_tc_tiling_on_sc=False),
        out_type=jax.ShapeDtypeStruct((NC * NPAD, _W), jnp.float32),
        scratch_types=[
            pltpu.VMEM((_DCH,), jnp.int32),
            pltpu.VMEM((_DTAIL,), jnp.int32),
            pltpu.VMEM((_DCH, _W), jnp.float32),
            pltpu.VMEM((_DTAIL, _W), jnp.float32),
            pltpu.VMEM((_ZR, _W), jnp.float32),
            pltpu.VMEM_SHARED((NPAD, _W), jnp.float32),
        ],
    )


def _deg_kernel(col):
    return _build_deg()(col)


_CH = 96
_CPB = 30
_NBLK = 7
_CPT = _CPB * _NBLK
_EPT = _CPT * _CH
_EPAD = NS * _EPT
_ROWS_PW = NPAD // NS
_HW = D // 2


def _agg_body(row_hbm, col_hbm, g00, g01, g10, g11, out0_hbm, out1_hbm,
              ridxb, cidxb, rows0, rows1, accum, sem0, sem1):
    c = lax.axis_index("c")
    s = lax.axis_index("s")
    zeros16 = jnp.zeros((16,), jnp.float32)

    def gather(g, j, rows, sem):
        pltpu.async_copy(g.at[ridxb.at[j]], rows, sem)

    def gwait(g, rows, sem):
        pltpu.make_async_copy(g.at[pl.ds(0, _CH)], rows, sem).wait()

    for half in range(2):
        ga = g00 if half == 0 else g01
        gb = g10 if half == 0 else g11
        out = out0_hbm if half == 0 else out1_hbm

        def zr(i, _):
            for j in range(_HW // 16):
                rows0[i, pl.ds(j * 16, 16)] = zeros16
            return 0

        lax.fori_loop(0, _CH, zr, 0)

        def zacc(k, _):
            pltpu.sync_copy(rows0, accum.at[pl.ds(s * _ROWS_PW + k * _CH, _CH)])
            return 0

        lax.fori_loop(0, _ROWS_PW // _CH, zacc, 0)
        pltpu.sync_copy(rows0.at[pl.ds(0, _ROWS_PW - (_ROWS_PW // _CH) * _CH)],
                        accum.at[pl.ds(s * _ROWS_PW + (_ROWS_PW // _CH) * _CH,
                                       _ROWS_PW - (_ROWS_PW // _CH) * _CH)])
        plsc.subcore_barrier()

        def blk_body(b, _):
            base = s * _CPT + b * _CPB
            pltpu.sync_copy(row_hbm.at[pl.ds(base, _CPB)], ridxb)
            pltpu.sync_copy(col_hbm.at[pl.ds(base, _CPB)], cidxb)

            def run(g):
                gather(g, 0, rows0, sem0)
                gather(g, 1, rows1, sem1)

                def pair(p, _):
                    j = 2 * p
                    gwait(g, rows0, sem0)
                    pltpu.sync_copy(rows0, accum.at[cidxb.at[j]], add=True)
                    gather(g, j + 2, rows0, sem0)
                    gwait(g, rows1, sem1)
                    pltpu.sync_copy(rows1, accum.at[cidxb.at[j + 1]], add=True)
                    gather(g, j + 3, rows1, sem1)
                    return 0

                lax.fori_loop(0, _CPB // 2 - 1, pair, 0)
                gwait(g, rows0, sem0)
                pltpu.sync_copy(rows0, accum.at[cidxb.at[_CPB - 2]], add=True)
                gwait(g, rows1, sem1)
                pltpu.sync_copy(rows1, accum.at[cidxb.at[_CPB - 1]], add=True)

            @pl.when(c == 0)
            def _():
                run(ga)

            @pl.when(c == 1)
            def _():
                run(gb)

            return 0

        lax.fori_loop(0, _NBLK, blk_body, 0)

        plsc.subcore_barrier()
        pltpu.sync_copy(accum.at[pl.ds(s * _ROWS_PW, _ROWS_PW)],
                        out.at[pl.ds(c * NPAD + s * _ROWS_PW, _ROWS_PW)])
        plsc.subcore_barrier()


@functools.cache
def _build_agg():
    return pl.kernel(
        _agg_body,
        mesh=plsc.VectorSubcoreMesh(**_MESH),
        compiler_params=pltpu.CompilerParams(needs_layout_passes=False, use_tc_tiling_on_sc=False),
        out_type=(jax.ShapeDtypeStruct((NC * NPAD, _HW), jnp.float32),
                  jax.ShapeDtypeStruct((NC * NPAD, _HW), jnp.float32)),
        scratch_types=[
            pltpu.VMEM((_CPB, _CH), jnp.int32),
            pltpu.VMEM((_CPB, _CH), jnp.int32),
            pltpu.VMEM((_CH, _HW), jnp.float32),
            pltpu.VMEM((_CH, _HW), jnp.float32),
            pltpu.VMEM_SHARED((NPAD, _HW), jnp.float32),
            pltpu.SemaphoreType.DMA,
            pltpu.SemaphoreType.DMA,
        ],
    )


def _agg_kernel(row2d, col2d, g00, g01, g10, g11):
    return _build_agg()(row2d, col2d, g00, g01, g10, g11)


_BR = 2000
_GRID = N // _BR


def _rowspec(w=D):
    return pl.BlockSpec((_BR, w), lambda i: (i, 0))


def _fullspec(shape):
    return pl.BlockSpec(shape, lambda i: tuple(0 for _ in shape))


def _tc1_body(d0_ref, d1_ref, x_ref, wc_ref,
              g00_ref, g01_ref, g10_ref, g11_ref, dinv_ref):
    deg16 = d0_ref[...] + d1_ref[...]
    deg = jnp.concatenate([deg16] * (D // _W), axis=1)
    dinv = jnp.where(deg > 0, lax.rsqrt(jnp.maximum(deg, 1e-12)), 0.0)
    dinv_ref[...] = dinv
    h = jnp.dot(x_ref[...], wc_ref[...], preferred_element_type=jnp.float32,
                precision=lax.Precision.HIGHEST)
    ge = h[:, :D] * dinv
    gd = h[:, D:] * dinv
    g00_ref[...] = ge[:, :_HW]
    g01_ref[...] = ge[:, _HW:]
    g10_ref[...] = gd[:, :_HW]
    g11_ref[...] = gd[:, _HW:]


def _tc1(d0, d1, x, wc):
    return pl.pallas_call(
        _tc1_body,
        grid=(_GRID,),
        in_specs=[_rowspec(_W), _rowspec(_W), _rowspec(), _fullspec((D, 2 * D))],
        out_specs=(_rowspec(_HW), _rowspec(_HW), _rowspec(_HW), _rowspec(_HW),
                   _rowspec()),
        out_shape=(jax.ShapeDtypeStruct((N, _HW), jnp.float32),
                   jax.ShapeDtypeStruct((N, _HW), jnp.float32),
                   jax.ShapeDtypeStruct((N, _HW), jnp.float32),
                   jax.ShapeDtypeStruct((N, _HW), jnp.float32),
                   jax.ShapeDtypeStruct((N, D), jnp.float32)),
        compiler_params=pltpu.CompilerParams(
            dimension_semantics=("arbitrary",)),
    )(d0, d1, x, wc)


def _tc2_body(s0l_ref, s0h_ref, s1l_ref, s1h_ref, dinv_ref,
              b1e_ref, b1d_ref, w2e_ref, w2d_ref,
              g00_ref, g01_ref, g10_ref, g11_ref, x1e_ref, x1d_ref):
    dinv = dinv_ref[...]
    s0 = jnp.concatenate([s0l_ref[...], s0h_ref[...]], axis=1)
    s1 = jnp.concatenate([s1l_ref[...], s1h_ref[...]], axis=1)
    x1e = jnp.tanh(s0 * dinv + b1e_ref[...])
    x1d = jnp.tanh(s1 * dinv + b1d_ref[...])
    x1e_ref[...] = x1e
    x1d_ref[...] = x1d
    ge = dinv * jnp.dot(x1e, w2e_ref[...], preferred_element_type=jnp.float32,
                        precision=lax.Precision.HIGHEST)
    gd = dinv * jnp.dot(x1d, w2d_ref[...], preferred_element_type=jnp.float32,
                        precision=lax.Precision.HIGHEST)
    g00_ref[...] = ge[:, :_HW]
    g01_ref[...] = ge[:, _HW:]
    g10_ref[...] = gd[:, :_HW]
    g11_ref[...] = gd[:, _HW:]


def _tc2(s0l, s0h, s1l, s1h, dinv, b1e, b1d, w2e, w2d):
    return pl.pallas_call(
        _tc2_body,
        grid=(_GRID,),
        in_specs=[_rowspec(_HW), _rowspec(_HW), _rowspec(_HW), _rowspec(_HW),
                  _rowspec(),
                  _fullspec((1, D)), _fullspec((1, D)),
                  _fullspec((D, D)), _fullspec((D, D))],
        out_specs=(_rowspec(_HW), _rowspec(_HW), _rowspec(_HW), _rowspec(_HW),
                   _rowspec(), _rowspec()),
        out_shape=(jax.ShapeDtypeStruct((N, _HW), jnp.float32),
                   jax.ShapeDtypeStruct((N, _HW), jnp.float32),
                   jax.ShapeDtypeStruct((N, _HW), jnp.float32),
                   jax.ShapeDtypeStruct((N, _HW), jnp.float32),
                   jax.ShapeDtypeStruct((N, D), jnp.float32),
                   jax.ShapeDtypeStruct((N, D), jnp.float32)),
        compiler_params=pltpu.CompilerParams(
            dimension_semantics=("arbitrary",)),
    )(s0l, s0h, s1l, s1h, dinv, b1e, b1d, w2e, w2d)


def _layernorm(y, w, b, eps=1e-5):
    mu = jnp.mean(y, axis=-1, keepdims=True)
    var = jnp.mean((y - mu) ** 2, axis=-1, keepdims=True)
    return (y - mu) / jnp.sqrt(var + eps) * w + b


def _tc3_body(s0l_ref, s0h_ref, s1l_ref, s1h_ref, dinv_ref, x1e_ref, x1d_ref,
              b2e_ref, b2d_ref, lwe_ref, lbe_ref, lwd_ref, lbd_ref,
              lnwe_ref, lnbe_ref, lnwd_ref, lnbd_ref,
              bnw_ref, bnb_ref, fw_ref, out_ref):
    dinv = dinv_ref[...]
    s0 = jnp.concatenate([s0l_ref[...], s0h_ref[...]], axis=1)
    s1 = jnp.concatenate([s1l_ref[...], s1h_ref[...]], axis=1)
    x2e = jnp.tanh(s0 * dinv + b2e_ref[...])
    x2d = jnp.tanh(s1 * dinv + b2d_ref[...])
    xme = jnp.maximum(x1e_ref[...], x2e)
    xmd = jnp.maximum(x1d_ref[...], x2d)
    ye = jnp.dot(xme, lwe_ref[...], preferred_element_type=jnp.float32,
                 precision=lax.Precision.HIGHEST) + lbe_ref[...]
    yd = jnp.dot(xmd, lwd_ref[...], preferred_element_type=jnp.float32,
                 precision=lax.Precision.HIGHEST) + lbd_ref[...]
    lne = _layernorm(ye, lnwe_ref[...], lnbe_ref[...])
    lnd = _layernorm(yd, lnwd_ref[...], lnbd_ref[...])
    node = (lne + lnd) * 0.5
    node = node / jnp.sqrt(1.0 + 1e-5) * bnw_ref[...] + bnb_ref[...]
    out_ref[...] = jnp.sum(node * fw_ref[...], axis=1).reshape(1, 8, _BR // 8)


def _tc3(s0l, s0h, s1l, s1h, dinv, x1e, x1d, b2e, b2d, lwe, lbe, lwd, lbd,
         lnwe, lnbe, lnwd, lnbd, bnw, bnb, fw):
    return pl.pallas_call(
        _tc3_body,
        grid=(_GRID,),
        in_specs=[_rowspec(_HW), _rowspec(_HW), _rowspec(_HW), _rowspec(_HW),
                  _rowspec(), _rowspec(), _rowspec(),
                  _fullspec((1, D)), _fullspec((1, D)),
                  _fullspec((D, D)), _fullspec((1, D)),
                  _fullspec((D, D)), _fullspec((1, D)),
                  _fullspec((1, D)), _fullspec((1, D)),
                  _fullspec((1, D)), _fullspec((1, D)),
                  _fullspec((1, D)), _fullspec((1, D)),
                  _fullspec((1, D))],
        out_specs=pl.BlockSpec((1, 8, _BR // 8), lambda i: (i, 0, 0)),
        out_shape=jax.ShapeDtypeStruct((_GRID, 8, _BR // 8), jnp.float32),
        compiler_params=pltpu.CompilerParams(
            dimension_semantics=("arbitrary",)),
    )(s0l, s0h, s1l, s1h, dinv, x1e, x1d, b2e, b2d, lwe, lbe, lwd, lbd,
      lnwe, lnbe, lnwd, lnbd, bnw, bnb, fw).reshape(N)


def kernel(x, edge_index, enc_W1, enc_b1, enc_W2, enc_b2, enc_linW, enc_linb,
           dec_W1, dec_b1, dec_W2, dec_b2, dec_linW, dec_linb,
           enc_ln_w, enc_ln_b, dec_ln_w, dec_ln_b, bn_w, bn_b, final_W):
    row = edge_index[0]
    col = edge_index[1]
    npad_e = _EPAD - E
    row_p = jnp.concatenate(
        [row, jnp.zeros((npad_e,), jnp.int32)]).reshape(NS * _CPT, _CH)
    col_p = jnp.concatenate(
        [col, jnp.full((npad_e,), N, jnp.int32)]).reshape(NS * _CPT, _CH)

    degp = _deg_kernel(col)
    d0 = degp[:N]
    d1 = degp[NPAD:NPAD + N]

    wc = jnp.concatenate([enc_W1, dec_W1], axis=1)
    g00, g01, g10, g11, dinv = _tc1(d0, d1, x, wc)

    r1 = lambda v: v.reshape(1, D)
    s1l, s1h = _agg_kernel(row_p, col_p, g00, g01, g10, g11)
    h00, h01, h10, h11, x1e, x1d = _tc2(
        s1l[:N], s1h[:N], s1l[NPAD:NPAD + N], s1h[NPAD:NPAD + N], dinv,
        r1(enc_b1), r1(dec_b1), enc_W2, dec_W2)

    s2l, s2h = _agg_kernel(row_p, col_p, h00, h01, h10, h11)
    return _tc3(s2l[:N], s2h[:N], s2l[NPAD:NPAD + N], s2h[NPAD:NPAD + N],
                dinv, x1e, x1d,
                r1(enc_b2), r1(dec_b2), enc_linW, r1(enc_linb),
                dec_linW, r1(dec_linb),
                r1(enc_ln_w), r1(enc_ln_b), r1(dec_ln_w), r1(dec_ln_b),
                r1(bn_w), r1(bn_b), final_W.reshape(1, D))


# speedup vs baseline: 1.2523x; 1.1067x over previous
import functools

import jax
import jax.numpy as jnp
from jax import lax
from jax.experimental import pallas as pl
from jax.experimental.pallas import tpu as pltpu
from jax.experimental.pallas import tpu_sc as plsc

N = 10000
E = 320000
D = 128
NC = 2
NS = 16
NPAD = 10240

_MESH = dict(core_axis_name="c", subcore_axis_name="s",
             num_cores=NC, num_subcores=NS)


_RPW = NPAD // NS
_W = 16
_ZR = 32
_DRPW = 105
_DBLK = 15


def _deg_body(col_hbm, out_hbm, cblk, ones, zbuf, shared):
    c = lax.axis_index("c")
    s = lax.axis_index("s")
    wid = c * NS + s
    zeros16 = jnp.zeros((_W,), jnp.float32)
    ones16 = jnp.ones((_W,), jnp.float32)

    def fill_ones(i, _):
        ones[i, pl.ds(0, _W)] = ones16
        return 0

    lax.fori_loop(0, _CH, fill_ones, 0)

    def zero_body(i, _):
        zbuf[i, pl.ds(0, _W)] = zeros16
        return 0

    lax.fori_loop(0, _ZR, zero_body, 0)

    def zcopy(k, _):
        pltpu.sync_copy(zbuf, shared.at[pl.ds(s * _RPW + k * _ZR, _ZR)])
        return 0

    lax.fori_loop(0, _RPW // _ZR, zcopy, 0)
    plsc.subcore_barrier()

    def blk_body(b, _):
        pltpu.sync_copy(col_hbm.at[pl.ds(wid * _DRPW + b * _DBLK, _DBLK)], cblk)

        def ch_body(j, _):
            pltpu.sync_copy(ones, shared.at[cblk.at[j]], add=True)
            return 0

        lax.fori_loop(0, _DBLK, ch_body, 0)
        return 0

    lax.fori_loop(0, _DRPW // _DBLK, blk_body, 0)
    plsc.subcore_barrier()

    pltpu.sync_copy(shared.at[pl.ds(s * _RPW, _RPW)],
                    out_hbm.at[pl.ds(c * NPAD + s * _RPW, _RPW)])


@functools.cache
def _build_deg():
    return pl.kernel(
        _deg_body,
        mesh=plsc.VectorSubcoreMesh(**_MESH),
        compiler_params=pltpu.CompilerParams(needs_layout_passes=False, use_tc_tiling_on_sc=False),
        out_type=jax.ShapeDtypeStruct((NC * NPAD, _W), jnp.float32),
        scratch_types=[
            pltpu.VMEM((_DBLK, _CH), jnp.int32),
            pltpu.VMEM((_CH, _W), jnp.float32),
            pltpu.VMEM((_ZR, _W), jnp.float32),
            pltpu.VMEM_SHARED((NPAD, _W), jnp.float32),
        ],
    )


def _deg_kernel(col2d):
    return _build_deg()(col2d)


_CH = 96
_CPB = 30
_NBLK = 7
_CPT = _CPB * _NBLK
_EPT = _CPT * _CH
_EPAD = NS * _EPT
_ROWS_PW = NPAD // NS
_HW = D // 2


def _agg_body(row_hbm, col_hbm, g00, g01, g10, g11, out0_hbm, out1_hbm,
              ridxb, cidxb, rows0, rows1, accum, sem0, sem1):
    c = lax.axis_index("c")
    s = lax.axis_index("s")
    zeros16 = jnp.zeros((16,), jnp.float32)

    def gather(g, j, rows, sem):
        pltpu.async_copy(g.at[ridxb.at[j]], rows, sem)

    def gwait(g, rows, sem):
        pltpu.make_async_copy(g.at[pl.ds(0, _CH)], rows, sem).wait()

    for half in range(2):
        ga = g00 if half == 0 else g01
        gb = g10 if half == 0 else g11
        out = out0_hbm if half == 0 else out1_hbm

        def zr(i, _):
            for j in range(_HW // 16):
                rows0[i, pl.ds(j * 16, 16)] = zeros16
            return 0

        lax.fori_loop(0, _CH, zr, 0)

        def zacc(k, _):
            pltpu.sync_copy(rows0, accum.at[pl.ds(s * _ROWS_PW + k * _CH, _CH)])
            return 0

        lax.fori_loop(0, _ROWS_PW // _CH, zacc, 0)
        pltpu.sync_copy(rows0.at[pl.ds(0, _ROWS_PW - (_ROWS_PW // _CH) * _CH)],
                        accum.at[pl.ds(s * _ROWS_PW + (_ROWS_PW // _CH) * _CH,
                                       _ROWS_PW - (_ROWS_PW // _CH) * _CH)])
        plsc.subcore_barrier()

        def blk_body(b, _):
            base = s * _CPT + b * _CPB
            pltpu.sync_copy(row_hbm.at[pl.ds(base, _CPB)], ridxb)
            pltpu.sync_copy(col_hbm.at[pl.ds(base, _CPB)], cidxb)

            def run(g):
                gather(g, 0, rows0, sem0)
                gather(g, 1, rows1, sem1)

                def pair(p, _):
                    j = 2 * p
                    gwait(g, rows0, sem0)
                    pltpu.sync_copy(rows0, accum.at[cidxb.at[j]], add=True)
                    gather(g, j + 2, rows0, sem0)
                    gwait(g, rows1, sem1)
                    pltpu.sync_copy(rows1, accum.at[cidxb.at[j + 1]], add=True)
                    gather(g, j + 3, rows1, sem1)
                    return 0

                lax.fori_loop(0, _CPB // 2 - 1, pair, 0)
                gwait(g, rows0, sem0)
                pltpu.sync_copy(rows0, accum.at[cidxb.at[_CPB - 2]], add=True)
                gwait(g, rows1, sem1)
                pltpu.sync_copy(rows1, accum.at[cidxb.at[_CPB - 1]], add=True)

            @pl.when(c == 0)
            def _():
                run(ga)

            @pl.when(c == 1)
            def _():
                run(gb)

            return 0

        lax.fori_loop(0, _NBLK, blk_body, 0)

        plsc.subcore_barrier()
        pltpu.sync_copy(accum.at[pl.ds(s * _ROWS_PW, _ROWS_PW)],
                        out.at[pl.ds(c * NPAD + s * _ROWS_PW, _ROWS_PW)])
        plsc.subcore_barrier()


@functools.cache
def _build_agg():
    return pl.kernel(
        _agg_body,
        mesh=plsc.VectorSubcoreMesh(**_MESH),
        compiler_params=pltpu.CompilerParams(needs_layout_passes=False, use_tc_tiling_on_sc=False),
        out_type=(jax.ShapeDtypeStruct((NC * NPAD, _HW), jnp.float32),
                  jax.ShapeDtypeStruct((NC * NPAD, _HW), jnp.float32)),
        scratch_types=[
            pltpu.VMEM((_CPB, _CH), jnp.int32),
            pltpu.VMEM((_CPB, _CH), jnp.int32),
            pltpu.VMEM((_CH, _HW), jnp.float32),
            pltpu.VMEM((_CH, _HW), jnp.float32),
            pltpu.VMEM_SHARED((NPAD, _HW), jnp.float32),
            pltpu.SemaphoreType.DMA,
            pltpu.SemaphoreType.DMA,
        ],
    )


def _agg_kernel(row2d, col2d, g00, g01, g10, g11):
    return _build_agg()(row2d, col2d, g00, g01, g10, g11)


_BR = 2048
_GRID = NPAD // _BR
_OFF = NPAD // _BR


def _rowspec(w=D, off=0):
    return pl.BlockSpec((_BR, w), lambda i: (i + off, 0))


def _fullspec(shape):
    return pl.BlockSpec(shape, lambda i: tuple(0 for _ in shape))


def _tc1_body(d0_ref, d1_ref, x_ref, wc_ref,
              g00_ref, g01_ref, g10_ref, g11_ref, dinv_ref):
    deg16 = d0_ref[...] + d1_ref[...]
    deg = jnp.concatenate([deg16] * (D // _W), axis=1)
    dinv = jnp.where(deg > 0, lax.rsqrt(jnp.maximum(deg, 1e-12)), 0.0)
    dinv_ref[...] = dinv
    h = jnp.dot(x_ref[...], wc_ref[...], preferred_element_type=jnp.float32,
                precision=lax.Precision.HIGHEST)
    ge = h[:, :D] * dinv
    gd = h[:, D:] * dinv
    g00_ref[...] = ge[:, :_HW]
    g01_ref[...] = ge[:, _HW:]
    g10_ref[...] = gd[:, :_HW]
    g11_ref[...] = gd[:, _HW:]


def _tc1(degp, x, wc):
    return pl.pallas_call(
        _tc1_body,
        grid=(_GRID,),
        in_specs=[_rowspec(_W), _rowspec(_W, _OFF), _rowspec(),
                  _fullspec((D, 2 * D))],
        out_specs=(_rowspec(_HW), _rowspec(_HW), _rowspec(_HW), _rowspec(_HW),
                   _rowspec()),
        out_shape=(jax.ShapeDtypeStruct((NPAD, _HW), jnp.float32),
                   jax.ShapeDtypeStruct((NPAD, _HW), jnp.float32),
                   jax.ShapeDtypeStruct((NPAD, _HW), jnp.float32),
                   jax.ShapeDtypeStruct((NPAD, _HW), jnp.float32),
                   jax.ShapeDtypeStruct((NPAD, D), jnp.float32)),
        compiler_params=pltpu.CompilerParams(
            dimension_semantics=("arbitrary",)),
    )(degp, degp, x, wc)


def _tc2_body(s0l_ref, s0h_ref, s1l_ref, s1h_ref, dinv_ref,
              b1e_ref, b1d_ref, w2e_ref, w2d_ref,
              g00_ref, g01_ref, g10_ref, g11_ref, x1e_ref, x1d_ref):
    dinv = dinv_ref[...]
    s0 = jnp.concatenate([s0l_ref[...], s0h_ref[...]], axis=1)
    s1 = jnp.concatenate([s1l_ref[...], s1h_ref[...]], axis=1)
    x1e = jnp.tanh(s0 * dinv + b1e_ref[...])
    x1d = jnp.tanh(s1 * dinv + b1d_ref[...])
    x1e_ref[...] = x1e
    x1d_ref[...] = x1d
    ge = dinv * jnp.dot(x1e, w2e_ref[...], preferred_element_type=jnp.float32,
                        precision=lax.Precision.HIGHEST)
    gd = dinv * jnp.dot(x1d, w2d_ref[...], preferred_element_type=jnp.float32,
                        precision=lax.Precision.HIGHEST)
    g00_ref[...] = ge[:, :_HW]
    g01_ref[...] = ge[:, _HW:]
    g10_ref[...] = gd[:, :_HW]
    g11_ref[...] = gd[:, _HW:]


def _tc2(sl, sh, dinv, b1e, b1d, w2e, w2d):
    return pl.pallas_call(
        _tc2_body,
        grid=(_GRID,),
        in_specs=[_rowspec(_HW), _rowspec(_HW),
                  _rowspec(_HW, _OFF), _rowspec(_HW, _OFF),
                  _rowspec(),
                  _fullspec((1, D)), _fullspec((1, D)),
                  _fullspec((D, D)), _fullspec((D, D))],
        out_specs=(_rowspec(_HW), _rowspec(_HW), _rowspec(_HW), _rowspec(_HW),
                   _rowspec(), _rowspec()),
        out_shape=(jax.ShapeDtypeStruct((NPAD, _HW), jnp.float32),
                   jax.ShapeDtypeStruct((NPAD, _HW), jnp.float32),
                   jax.ShapeDtypeStruct((NPAD, _HW), jnp.float32),
                   jax.ShapeDtypeStruct((NPAD, _HW), jnp.float32),
                   jax.ShapeDtypeStruct((NPAD, D), jnp.float32),
                   jax.ShapeDtypeStruct((NPAD, D), jnp.float32)),
        compiler_params=pltpu.CompilerParams(
            dimension_semantics=("arbitrary",)),
    )(sl, sh, sl, sh, dinv, b1e, b1d, w2e, w2d)


def _layernorm(y, w, b, eps=1e-5):
    mu = jnp.mean(y, axis=-1, keepdims=True)
    var = jnp.mean((y - mu) ** 2, axis=-1, keepdims=True)
    return (y - mu) / jnp.sqrt(var + eps) * w + b


def _tc3_body(s0l_ref, s0h_ref, s1l_ref, s1h_ref, dinv_ref, x1e_ref, x1d_ref,
              b2e_ref, b2d_ref, lwe_ref, lbe_ref, lwd_ref, lbd_ref,
              lnwe_ref, lnbe_ref, lnwd_ref, lnbd_ref,
              bnw_ref, bnb_ref, fw_ref, out_ref):
    dinv = dinv_ref[...]
    s0 = jnp.concatenate([s0l_ref[...], s0h_ref[...]], axis=1)
    s1 = jnp.concatenate([s1l_ref[...], s1h_ref[...]], axis=1)
    x2e = jnp.tanh(s0 * dinv + b2e_ref[...])
    x2d = jnp.tanh(s1 * dinv + b2d_ref[...])
    xme = jnp.maximum(x1e_ref[...], x2e)
    xmd = jnp.maximum(x1d_ref[...], x2d)
    ye = jnp.dot(xme, lwe_ref[...], preferred_element_type=jnp.float32,
                 precision=lax.Precision.HIGHEST) + lbe_ref[...]
    yd = jnp.dot(xmd, lwd_ref[...], preferred_element_type=jnp.float32,
                 precision=lax.Precision.HIGHEST) + lbd_ref[...]
    lne = _layernorm(ye, lnwe_ref[...], lnbe_ref[...])
    lnd = _layernorm(yd, lnwd_ref[...], lnbd_ref[...])
    node = (lne + lnd) * 0.5
    node = node / jnp.sqrt(1.0 + 1e-5) * bnw_ref[...] + bnb_ref[...]
    out_ref[...] = jnp.sum(node * fw_ref[...], axis=1).reshape(1, 8, _BR // 8)


def _tc3(sl, sh, dinv, x1e, x1d, b2e, b2d, lwe, lbe, lwd, lbd,
         lnwe, lnbe, lnwd, lnbd, bnw, bnb, fw):
    return pl.pallas_call(
        _tc3_body,
        grid=(_GRID,),
        in_specs=[_rowspec(_HW), _rowspec(_HW),
                  _rowspec(_HW, _OFF), _rowspec(_HW, _OFF),
                  _rowspec(), _rowspec(), _rowspec(),
                  _fullspec((1, D)), _fullspec((1, D)),
                  _fullspec((D, D)), _fullspec((1, D)),
                  _fullspec((D, D)), _fullspec((1, D)),
                  _fullspec((1, D)), _fullspec((1, D)),
                  _fullspec((1, D)), _fullspec((1, D)),
                  _fullspec((1, D)), _fullspec((1, D)),
                  _fullspec((1, D))],
        out_specs=pl.BlockSpec((1, 8, _BR // 8), lambda i: (i, 0, 0)),
        out_shape=jax.ShapeDtypeStruct((_GRID, 8, _BR // 8), jnp.float32),
        compiler_params=pltpu.CompilerParams(
            dimension_semantics=("arbitrary",)),
    )(sl, sh, sl, sh, dinv, x1e, x1d, b2e, b2d, lwe, lbe, lwd, lbd,
      lnwe, lnbe, lnwd, lnbd, bnw, bnb, fw).reshape(NPAD)[:N]


def kernel(x, edge_index, enc_W1, enc_b1, enc_W2, enc_b2, enc_linW, enc_linb,
           dec_W1, dec_b1, dec_W2, dec_b2, dec_linW, dec_linb,
           enc_ln_w, enc_ln_b, dec_ln_w, dec_ln_b, bn_w, bn_b, final_W):
    row = edge_index[0]
    col = edge_index[1]
    npad_e = _EPAD - E
    row_p = jnp.concatenate(
        [row, jnp.zeros((npad_e,), jnp.int32)]).reshape(NS * _CPT, _CH)
    col_p = jnp.concatenate(
        [col, jnp.full((npad_e,), N, jnp.int32)]).reshape(NS * _CPT, _CH)
    x_p = jnp.concatenate([x, jnp.zeros((NPAD - N, D), x.dtype)], axis=0)

    degp = _deg_kernel(col_p)
    wc = jnp.concatenate([enc_W1, dec_W1], axis=1)
    g00, g01, g10, g11, dinv = _tc1(degp, x_p, wc)

    r1 = lambda v: v.reshape(1, D)
    s1l, s1h = _agg_kernel(row_p, col_p, g00, g01, g10, g11)
    h00, h01, h10, h11, x1e, x1d = _tc2(
        s1l, s1h, dinv, r1(enc_b1), r1(dec_b1), enc_W2, dec_W2)

    s2l, s2h = _agg_kernel(row_p, col_p, h00, h01, h10, h11)
    return _tc3(s2l, s2h, dinv, x1e, x1d,
                r1(enc_b2), r1(dec_b2), enc_linW, r1(enc_linb),
                dec_linW, r1(dec_linb),
                r1(enc_ln_w), r1(enc_ln_b), r1(dec_ln_w), r1(dec_ln_b),
                r1(bn_w), r1(bn_b), final_W.reshape(1, D))
